# Initial kernel scaffold; baseline (speedup 1.0000x reference)
import jax
import jax.numpy as jnp
from jax.experimental import pallas as pl


def kernel(u, v, n, gmf_u_emb, gmf_v_emb, u_emb, v_emb, W1, b1, W2, b2, Wp, bp):
    raise NotImplementedError("write your pallas kernel here")


# speedup vs baseline: 1.2582x; 1.2582x over previous
import functools

import jax
import jax.numpy as jnp
from jax import lax
from jax.experimental import pallas as pl
from jax.experimental.pallas import tpu as pltpu
from jax.experimental.pallas import tpu_sc as plsc

_NC, _NS = 2, 16
_NW = _NC * _NS
_EMB = 32
_CHUNK = 128


def _sc_gather(u2, v2, n2, gmf_u, gmf_v, u_t, v_t, B, BN):
    cb = B // _NW
    cn = BN // _NW
    kb = cb // _CHUNK
    kn = cn // _CHUNK
    mesh = plsc.VectorSubcoreMesh(core_axis_name="c", subcore_axis_name="s",
                                  num_cores=_NC, num_subcores=_NS)
    out_type = tuple(
        jax.ShapeDtypeStruct((sz, _EMB), jnp.float32)
        for sz in (B, B, B, B, BN, BN)
    )
    scratch = [
        pltpu.VMEM((kb, _CHUNK), jnp.int32),
        pltpu.VMEM((kb, _CHUNK), jnp.int32),
        pltpu.VMEM((kn, _CHUNK), jnp.int32),
        pltpu.VMEM((cn, _EMB), jnp.float32),
        pltpu.SemaphoreType.DMA,
    ]

    @functools.partial(pl.kernel, mesh=mesh, out_type=out_type,
                       scratch_types=scratch,
                       compiler_params=pltpu.CompilerParams(
                           use_tc_tiling_on_sc=False))
    def k(u_h, v_h, n_h, gmfu_h, gmfv_h, ut_h, vt_h,
          gu_o, gv_o, ue_o, ve_o, gn_o, ne_o,
          ui_v, vi_v, ni_v, rows_v, sem):
        w = lax.axis_index("s") * _NC + lax.axis_index("c")
        pltpu.sync_copy(u_h.at[w], ui_v)
        pltpu.sync_copy(v_h.at[w], vi_v)
        pltpu.sync_copy(n_h.at[w], ni_v)

        def do_set(tbl, idx2, nchunks, out, base, cnt):
            descs = []
            for i in range(nchunks):
                descs.append(pltpu.async_copy(
                    tbl.at[idx2.at[i]],
                    rows_v.at[pl.ds(i * _CHUNK, _CHUNK)], sem))
            for d in descs:
                d.wait()
            pltpu.sync_copy(rows_v.at[pl.ds(0, cnt)], out.at[pl.ds(base, cnt)])

        bu = w * cb
        bn = w * cn
        do_set(gmfu_h, ui_v, kb, gu_o, bu, cb)
        do_set(gmfv_h, vi_v, kb, gv_o, bu, cb)
        do_set(ut_h, ui_v, kb, ue_o, bu, cb)
        do_set(vt_h, vi_v, kb, ve_o, bu, cb)
        do_set(gmfv_h, ni_v, kn, gn_o, bn, cn)
        do_set(vt_h, ni_v, kn, ne_o, bn, cn)

    return k(u2, v2, n2, gmf_u, gmf_v, u_t, v_t)


def _tc_body(gu_r, gv_r, ue_r, ve_r, gn_r, ne_r,
             w1t_r, w1b_r, w2_r, wp1_r, wp2_r, b1_r, b2_r, bp_r,
             po_r, pn_r):
    w1b = w1b_r[...]
    w2 = w2_r[...]
    wp1 = wp1_r[...]
    wp2 = wp2_r[...]
    b1 = b1_r[...]
    b2 = b2_r[...]
    bp = bp_r[0, 0]
    gu = gu_r[...]
    au = jnp.dot(ue_r[...], w1t_r[...], preferred_element_type=jnp.float32)

    def head(a_u, other, gmf_prod):
        h1 = jnp.maximum(
            a_u + jnp.dot(other, w1b, preferred_element_type=jnp.float32) + b1,
            0.0)
        h2 = jnp.maximum(
            jnp.dot(h1, w2, preferred_element_type=jnp.float32) + b2, 0.0)
        return (jnp.sum(gmf_prod * wp1, axis=1)
                + jnp.sum(h2 * wp2, axis=1) + bp)

    po_r[:] = head(au, ve_r[...], gu * gv_r[...])
    for j in range(4):
        pn_r[j, :] = head(au, ne_r[j], gu * gn_r[j])


def _tc_dense(gu, gv, ue, ve, gn, ne, w1t, w1b, w2, wp1, wp2, b1, b2, bp):
    B = gu.shape[0]
    blk = 2048
    nb = B // blk
    row_spec = pl.BlockSpec((blk, _EMB), lambda i: (i, 0))
    neg_spec = pl.BlockSpec((4, blk, _EMB), lambda i: (0, i, 0))

    def full(a):
        return pl.BlockSpec(a.shape, lambda i: (0,) * a.ndim)

    out = pl.pallas_call(
        _tc_body,
        grid=(nb,),
        in_specs=[row_spec, row_spec, row_spec, row_spec, neg_spec, neg_spec,
                  full(w1t), full(w1b), full(w2), full(wp1), full(wp2),
                  full(b1), full(b2), full(bp)],
        out_specs=[pl.BlockSpec((blk,), lambda i: (i,)),
                   pl.BlockSpec((4, blk), lambda i: (0, i))],
        out_shape=[jax.ShapeDtypeStruct((B,), jnp.float32),
                   jax.ShapeDtypeStruct((4, B), jnp.float32)],
    )(gu, gv, ue, ve, gn.reshape(4, B, _EMB), ne.reshape(4, B, _EMB),
      w1t, w1b, w2, wp1, wp2, b1, b2, bp)
    return out


def kernel(u, v, n, gmf_u_emb, gmf_v_emb, u_emb, v_emb, W1, b1, W2, b2, Wp, bp):
    B = u.shape[0]
    nneg = n.shape[1]
    BN = B * nneg
    cb = B // _NW
    cn = BN // _NW
    u2 = u.astype(jnp.int32).reshape(_NW, cb // _CHUNK, _CHUNK)
    v2 = v.astype(jnp.int32).reshape(_NW, cb // _CHUNK, _CHUNK)
    n2 = n.astype(jnp.int32).T.reshape(_NW, cn // _CHUNK, _CHUNK)

    gu, gv, ue, ve, gn, ne = _sc_gather(
        u2, v2, n2, gmf_u_emb, gmf_v_emb, u_emb, v_emb, B, BN)

    w1t, w1b = W1[:_EMB], W1[_EMB:]
    wp1 = Wp[:_EMB, 0].reshape(1, _EMB)
    wp2 = Wp[_EMB:, 0].reshape(1, 16)
    po, pn = _tc_dense(gu, gv, ue, ve, gn, ne, w1t, w1b, W2,
                       wp1, wp2, b1.reshape(1, _EMB), b2.reshape(1, 16),
                       bp.reshape(1, 1))
    pred = po
    pred_n = pn.T.reshape(-1)
    return (pred, pred_n)
